# Initial kernel scaffold; baseline (speedup 1.0000x reference)
import jax
import jax.numpy as jnp
from jax.experimental import pallas as pl


def kernel(x, conv1_w, conv1_b, conv2_w, conv2_b, conv3_w, conv3_b, conv4_w, conv4_b, conv5_w, conv5_b, conv6_w, conv6_b, conv7_w, conv7_b, conv8_w, conv8_b, dl_w, dl_b, fc_w, fc_b):
    raise NotImplementedError("write your pallas kernel here")


# speedup vs baseline: 3.5511x; 3.5511x over previous
import functools

import jax
import jax.numpy as jnp
from jax.experimental import pallas as pl
from jax.experimental.pallas import tpu as pltpu


def _conv_kernel(x_ref, w_ref, b_ref, o_ref, *, oh, ow, c):
    n = x_ref.shape[0]
    m = n * oh * ow
    x = x_ref[...]
    acc = None
    for kh in range(3):
        for kw in range(3):
            t = kh * 3 + kw
            a = x[:, kh:kh + oh, kw:kw + ow, :].reshape(m, c)
            d = jnp.dot(a, w_ref[t * c:(t + 1) * c, :],
                        preferred_element_type=jnp.float32)
            acc = d if acc is None else acc + d
    r = jnp.maximum(acc + b_ref[...], 0.0)
    o_ref[...] = r.reshape(n, oh, ow, o_ref.shape[-1]).astype(o_ref.dtype)


def _conv_tail_kernel(x_ref, w_ref, b_ref, dlw_ref, dlb_ref, fcw_ref,
                      fcb_ref, o_ref, *, c):
    n = x_ref.shape[0]
    m = n * 4
    x = x_ref[...]
    acc = None
    for kh in range(3):
        for kw in range(3):
            t = kh * 3 + kw
            a = x[:, kh:kh + 2, kw:kw + 2, :].reshape(m, c)
            d = jnp.dot(a, w_ref[t * c:(t + 1) * c, :],
                        preferred_element_type=jnp.float32)
            acc = d if acc is None else acc + d
    r = jnp.maximum(acc + b_ref[...], 0.0).astype(jnp.bfloat16)
    pooled = jnp.mean(r.reshape(n, 4, r.shape[-1]).astype(jnp.float32),
                      axis=1)
    h = jnp.dot(pooled.astype(jnp.bfloat16), dlw_ref[...],
                preferred_element_type=jnp.float32) + dlb_ref[...]
    logits = jnp.dot(h.astype(jnp.bfloat16), fcw_ref[...],
                     preferred_element_type=jnp.float32) + fcb_ref[...]
    o_ref[...] = logits.reshape(o_ref.shape)


def _vmem_limit(*arrays):
    need = 2 * sum(a.size * a.dtype.itemsize for a in arrays) + (6 << 20)
    return int(min(max(need, 32 << 20), 58 << 20))


def _conv(x, w, b, *, split):
    n, h, wd, c = x.shape
    cout = w.shape[1]
    oh, ow = h - 2, wd - 2
    if split == "batch":
        nb = n // 2
        in_specs = [
            pl.BlockSpec((nb, h, wd, c), lambda i: (i, 0, 0, 0)),
            pl.BlockSpec(w.shape, lambda i: (0, 0)),
            pl.BlockSpec((1, cout), lambda i: (0, 0)),
        ]
        out_spec = pl.BlockSpec((nb, oh, ow, cout), lambda i: (i, 0, 0, 0))
    else:
        tn = cout // 2
        in_specs = [
            pl.BlockSpec((n, h, wd, c), lambda i: (0, 0, 0, 0)),
            pl.BlockSpec((w.shape[0], tn), lambda i: (0, i)),
            pl.BlockSpec((1, tn), lambda i: (0, i)),
        ]
        out_spec = pl.BlockSpec((n, oh, ow, tn), lambda i: (0, 0, 0, i))
    return pl.pallas_call(
        functools.partial(_conv_kernel, oh=oh, ow=ow, c=c),
        out_shape=jax.ShapeDtypeStruct((n, oh, ow, cout), jnp.bfloat16),
        grid=(2,),
        in_specs=in_specs,
        out_specs=out_spec,
        compiler_params=pltpu.CompilerParams(
            dimension_semantics=("parallel",),
            vmem_limit_bytes=_vmem_limit(x, w, b)),
    )(x, w, b)


def _conv_tail(x, w, b, dl_w, dl_b, fc_w, fc_b):
    n, h, wd, c = x.shape
    nb = n // 2
    out = pl.pallas_call(
        functools.partial(_conv_tail_kernel, c=c),
        out_shape=jax.ShapeDtypeStruct((2, nb, fc_w.shape[1]), jnp.float32),
        grid=(2,),
        in_specs=[
            pl.BlockSpec((nb, h, wd, c), lambda i: (i, 0, 0, 0)),
            pl.BlockSpec(w.shape, lambda i: (0, 0)),
            pl.BlockSpec((1, w.shape[1]), lambda i: (0, 0)),
            pl.BlockSpec(dl_w.shape, lambda i: (0, 0)),
            pl.BlockSpec(dl_b.shape, lambda i: (0, 0)),
            pl.BlockSpec(fc_w.shape, lambda i: (0, 0)),
            pl.BlockSpec(fc_b.shape, lambda i: (0, 0)),
        ],
        out_specs=pl.BlockSpec((1, nb, fc_w.shape[1]), lambda i: (i, 0, 0)),
        compiler_params=pltpu.CompilerParams(
            dimension_semantics=("parallel",),
            vmem_limit_bytes=_vmem_limit(x, w, dl_w)),
    )(x, w, b, dl_w, dl_b, fc_w, fc_b)
    return out.reshape(n, fc_w.shape[1])


def kernel(x, conv1_w, conv1_b, conv2_w, conv2_b, conv3_w, conv3_b,
           conv4_w, conv4_b, conv5_w, conv5_b, conv6_w, conv6_b,
           conv7_w, conv7_b, conv8_w, conv8_b, dl_w, dl_b, fc_w, fc_b):
    xh = jnp.transpose(x, (0, 2, 3, 1)).astype(jnp.bfloat16)
    cin = xh.shape[-1]
    cpad = 384
    xh = jnp.pad(xh, ((0, 0), (0, 0), (0, 0), (0, cpad - cin)))
    w1 = conv1_w[:9 * cin].reshape(9, cin, conv1_w.shape[1])
    w1 = jnp.pad(w1, ((0, 0), (0, cpad - cin), (0, 0)))
    w1 = w1.reshape(9 * cpad, conv1_w.shape[1])

    h = _conv(xh, w1, conv1_b, split="batch")
    h = _conv(h, conv2_w, conv2_b, split="batch")
    h = _conv(h, conv3_w, conv3_b, split="batch")
    h = _conv(h, conv4_w, conv4_b, split="cout")
    h = _conv(h, conv5_w, conv5_b, split="cout")
    h = _conv(h, conv6_w, conv6_b, split="cout")
    h = _conv(h, conv7_w, conv7_b, split="cout")
    logits = _conv_tail(h, conv8_w, conv8_b, dl_w, dl_b, fc_w, fc_b)
    return logits[:, :2]
